# Initial kernel scaffold; baseline (speedup 1.0000x reference)
import jax
import jax.numpy as jnp
from jax.experimental import pallas as pl


def kernel(x, edge_index, edge_weight, W1, b1, W2, b2):
    raise NotImplementedError("write your pallas kernel here")


# speedup vs baseline: 5.7429x; 5.7429x over previous
import jax
import jax.numpy as jnp
from jax import lax
from jax.experimental import pallas as pl
from jax.experimental.pallas import tpu as pltpu
from jax.experimental.pallas import tpu_sc as plsc

N = 10000
NPAD = 10240
E = 160000
EPAD = 163840
L = 16
NC, NS = 2, 16
CH = 128
EB = 128
ER = EPAD // EB
RPT = NPAD // NS
NB = 128
ERPT = ER // NS
ERPW = ER // (NC * NS)

_mesh = plsc.VectorSubcoreMesh(core_axis_name="c", subcore_axis_name="s",
                               num_cores=NC, num_subcores=NS)
_sc_params = pltpu.CompilerParams(needs_layout_passes=False)


def _splat_i(v):
    return jnp.full((L,), v, jnp.int32)


def _deg_body(col2, ewf, dega_out, degb_out,
              deg_sh, colb, ewb, nodev):
    c = lax.axis_index("c")
    s = lax.axis_index("s")
    t0 = s * RPT
    e0 = (c * NS + s) * ERPW

    pltpu.sync_copy(col2.at[pl.ds(e0, ERPW)], colb)
    pltpu.sync_copy(ewf.at[pl.ds(e0 * EB, ERPW * EB)], ewb)

    init = jnp.where(c == 0, 1.0, 0.0).astype(jnp.float32)

    def _init(k, z):
        nodev[pl.ds(k * L, L)] = jnp.full((L,), 1.0, jnp.float32) * init
        return z
    lax.fori_loop(0, RPT // L, _init, 0)
    pltpu.sync_copy(nodev, deg_sh.at[pl.ds(t0, RPT)])
    plsc.subcore_barrier()

    def _scat(b, z):
        pltpu.sync_copy(ewb.at[pl.ds(b * EB, EB)],
                        deg_sh.at[colb.at[b]], add=True)
        return z
    lax.fori_loop(0, ERPW, _scat, 0)
    plsc.subcore_barrier()

    pltpu.sync_copy(deg_sh.at[pl.ds(t0, RPT)], nodev)

    @pl.when(c == 0)
    def _():
        pltpu.sync_copy(nodev, dega_out.at[pl.ds(t0, RPT)])

    @pl.when(c == 1)
    def _():
        pltpu.sync_copy(nodev, degb_out.at[pl.ds(t0, RPT)])


_deg = pl.kernel(
    _deg_body,
    out_type=(jax.ShapeDtypeStruct((NPAD,), jnp.float32),
              jax.ShapeDtypeStruct((NPAD,), jnp.float32)),
    mesh=_mesh,
    compiler_params=_sc_params,
    scratch_types=[
        pltpu.VMEM_SHARED((NPAD,), jnp.float32),
        pltpu.VMEM((ERPW, EB), jnp.int32),
        pltpu.VMEM((ERPW * EB,), jnp.float32),
        pltpu.VMEM((RPT,), jnp.float32),
    ],
)


def _rsqrt_body(da_ref, db_ref, dis_ref, dis2_ref):
    d = da_ref[...] + db_ref[...]
    r = lax.rsqrt(d)
    dis_ref[...] = r
    dis2_ref[...] = r * r


_rsqrt = pl.pallas_call(
    _rsqrt_body,
    out_shape=[jax.ShapeDtypeStruct((NPAD // 128, 128), jnp.float32)] * 2,
)


def _norm_body(rowf, colf, ewf, dis, normf,
               dis_full, rown, coln, ewn, normb):
    c = lax.axis_index("c")
    s = lax.axis_index("s")
    n0 = (c * NS + s) * ERPW * EB

    pltpu.sync_copy(dis, dis_full)
    pltpu.sync_copy(rowf.at[pl.ds(n0, ERPW * EB)], rown)
    pltpu.sync_copy(colf.at[pl.ds(n0, ERPW * EB)], coln)
    pltpu.sync_copy(ewf.at[pl.ds(n0, ERPW * EB)], ewn)

    def _nrm(i, z):
        sl = pl.ds(i * L, L)
        ri = rown[sl]
        ci = coln[sl]
        wv = ewn[sl]
        dr = plsc.load_gather(dis_full, [ri])
        dc = plsc.load_gather(dis_full, [ci])
        normb[sl] = dr * wv * dc
        return z
    lax.fori_loop(0, ERPW * EB // L, _nrm, 0)
    pltpu.sync_copy(normb, normf.at[pl.ds(n0, ERPW * EB)])


_norm = pl.kernel(
    _norm_body,
    out_type=jax.ShapeDtypeStruct((EPAD,), jnp.float32),
    mesh=_mesh,
    compiler_params=_sc_params,
    scratch_types=[
        pltpu.VMEM((NPAD,), jnp.float32),
        pltpu.VMEM((ERPW * EB,), jnp.int32),
        pltpu.VMEM((ERPW * EB,), jnp.int32),
        pltpu.VMEM((ERPW * EB,), jnp.float32),
        pltpu.VMEM((ERPW * EB,), jnp.float32),
    ],
)


def _make_agg(nch):
    rounds = nch // NC

    def _body(*refs):
        ys = refs[:nch]
        packed, normf, dis2, bias = refs[nch:nch + 4]
        outs = refs[nch + 4:nch + 4 + nch]
        (acc, G0, G1, S0, S1, S2, S3, N0, N1, N2, N3, dis2v, biasv,
         sS0, sS1, sS2, sS3, gs0, gs1, sc0, sc1) = refs[nch + 4 + nch:]
        G = (G0, G1)
        S = (S0, S1, S2, S3)
        NRM = (N0, N1, N2, N3)
        sS = (sS0, sS1, sS2, sS3)
        gs = (gs0, gs1)
        sc = (sc0, sc1)

        c = lax.axis_index("c")
        s = lax.axis_index("s")
        t0 = s * RPT
        e0 = s * ERPT

        pltpu.sync_copy(dis2.at[pl.ds(t0, RPT)], dis2v)

        def _scale_rows(buf, nrows, base_fn):
            def _se(e, z):
                sp = base_fn(e)
                for k in range(CH // L):
                    sl = pl.ds(k * L, L)
                    buf[e, sl] = buf[e, sl] * sp
                return z
            lax.fori_loop(0, nrows, _se, 0)

        def _round(y, out, ch):
            def _didx(b, j):
                return pltpu.make_async_copy(packed.at[e0 + b], S[j], sS[j])

            def _dnrm(b, j):
                return pltpu.make_async_copy(
                    normf.at[pl.ds((e0 + b) * EB, EB)], NRM[j], sS[j])

            def _dgat(b, p, j):
                return pltpu.make_async_copy(y.at[S[j].at[0]], G[p], gs[p])

            def _dsct(p, j):
                return pltpu.make_async_copy(G[p], acc.at[S[j].at[1]], sc[p])

            pltpu.sync_copy(bias.at[pl.ds(ch * CH, CH)], biasv)
            for bb in range(RPT // NB):
                r0 = t0 + bb * NB
                pltpu.sync_copy(y.at[pl.ds(r0, NB)], G0)
                _scale_rows(G0, NB, lambda e, _bb=bb: plsc.load_gather(
                    dis2v, [_splat_i(_bb * NB + e)]))
                pltpu.sync_copy(G0, acc.at[pl.ds(r0, NB)])
            plsc.subcore_barrier()

            _didx(0, 0).start()
            _dnrm(0, 0).start()
            _didx(1, 1).start()
            _dnrm(1, 1).start()
            _didx(0, 0).wait()
            _dnrm(0, 0).wait()
            _dgat(0, 0, 0).start()

            def _step(d, z):
                for j in range(4):
                    b = d * 4 + j
                    pj, pj1 = j & 1, (j + 1) & 1
                    if j == 3:
                        @pl.when(b + 1 < ERPT)
                        def _():
                            _didx(b + 1, (j + 1) & 3).wait()
                            _dnrm(b + 1, (j + 1) & 3).wait()
                    else:
                        _didx(b + 1, (j + 1) & 3).wait()
                        _dnrm(b + 1, (j + 1) & 3).wait()
                    if j == 0:
                        @pl.when(b >= 1)
                        def _():
                            _dsct(pj1, (j + 3) & 3).wait()
                    else:
                        _dsct(pj1, (j + 3) & 3).wait()

                    @pl.when(b + 1 < ERPT)
                    def _():
                        _dgat(b + 1, pj1, (j + 1) & 3).start()

                    @pl.when(b + 2 < ERPT)
                    def _():
                        _didx(b + 2, (j + 2) & 3).start()
                        _dnrm(b + 2, (j + 2) & 3).start()
                    _dgat(b, pj, j & 3).wait()
                    _scale_rows(G[pj], EB,
                                lambda e, _j=j: plsc.load_gather(
                                    NRM[_j], [_splat_i(e)]))
                    _dsct(pj, j & 3).start(add=True)
                return z
            lax.fori_loop(0, ERPT // 4, _step, 0)
            _dsct(1, 3).wait()
            plsc.subcore_barrier()

            for bb in range(RPT // NB):
                r0 = t0 + bb * NB
                pltpu.sync_copy(acc.at[pl.ds(r0, NB)], G0)

                def _wo(e, z):
                    for k in range(CH // L):
                        sl = pl.ds(k * L, L)
                        G0[e, sl] = jnp.maximum(G0[e, sl] + biasv[sl], 0.0)
                    return z
                lax.fori_loop(0, NB, _wo, 0)
                pltpu.sync_copy(G0, out.at[pl.ds(r0, NB)])
            plsc.subcore_barrier()

        for core in range(NC):
            @pl.when(c == core)
            def _(core=core):
                for r in range(rounds):
                    ch = core * rounds + r
                    _round(ys[ch], outs[ch], ch)

    return pl.kernel(
        _body,
        out_type=tuple(jax.ShapeDtypeStruct((NPAD, CH), jnp.float32)
                       for _ in range(nch)),
        mesh=_mesh,
        compiler_params=_sc_params,
        scratch_types=[
            pltpu.VMEM_SHARED((NPAD, CH), jnp.float32),
            pltpu.VMEM((EB, CH), jnp.float32),
            pltpu.VMEM((EB, CH), jnp.float32),
            pltpu.VMEM((2, EB), jnp.int32),
            pltpu.VMEM((2, EB), jnp.int32),
            pltpu.VMEM((2, EB), jnp.int32),
            pltpu.VMEM((2, EB), jnp.int32),
            pltpu.VMEM((EB,), jnp.float32),
            pltpu.VMEM((EB,), jnp.float32),
            pltpu.VMEM((EB,), jnp.float32),
            pltpu.VMEM((EB,), jnp.float32),
            pltpu.VMEM((RPT,), jnp.float32),
            pltpu.VMEM((CH,), jnp.float32),
            pltpu.SemaphoreType.DMA,
            pltpu.SemaphoreType.DMA,
            pltpu.SemaphoreType.DMA,
            pltpu.SemaphoreType.DMA,
            pltpu.SemaphoreType.DMA,
            pltpu.SemaphoreType.DMA,
            pltpu.SemaphoreType.DMA,
            pltpu.SemaphoreType.DMA,
        ],
    )


_agg1 = _make_agg(4)
_agg2 = _make_agg(2)


def _mm1_body(x_ref, w_ref, *out_refs):
    acc = jnp.dot(x_ref[...], w_ref[...], preferred_element_type=jnp.float32)
    for ci in range(len(out_refs)):
        out_refs[ci][...] = acc[:, ci * CH:(ci + 1) * CH]


_mm1 = pl.pallas_call(
    _mm1_body,
    grid=(NPAD // 512,),
    in_specs=[pl.BlockSpec((512, 1728), lambda i: (i, 0)),
              pl.BlockSpec((1728, 512), lambda i: (0, 0))],
    out_specs=[pl.BlockSpec((512, CH), lambda i: (i, 0))] * 4,
    out_shape=[jax.ShapeDtypeStruct((NPAD, CH), jnp.float32)] * 4,
)


def _mm2_body(h0, h1, h2, h3, w_ref, *out_refs):
    h = jnp.concatenate([h0[...], h1[...], h2[...], h3[...]], axis=1)
    acc = jnp.dot(h, w_ref[...], preferred_element_type=jnp.float32)
    for ci in range(len(out_refs)):
        out_refs[ci][...] = acc[:, ci * CH:(ci + 1) * CH]


_mm2 = pl.pallas_call(
    _mm2_body,
    grid=(NPAD // 512,),
    in_specs=[pl.BlockSpec((512, CH), lambda i: (i, 0))] * 4 +
             [pl.BlockSpec((512, 256), lambda i: (0, 0))],
    out_specs=[pl.BlockSpec((512, CH), lambda i: (i, 0))] * 2,
    out_shape=[jax.ShapeDtypeStruct((NPAD, CH), jnp.float32)] * 2,
)


def kernel(x, edge_index, edge_weight, W1, b1, W2, b2):
    row = edge_index[0].astype(jnp.int32)
    col = edge_index[1].astype(jnp.int32)
    ew = edge_weight.astype(jnp.float32)
    rowp = jnp.pad(row, (0, EPAD - E))
    colp = jnp.pad(col, (0, EPAD - E))
    ewp = jnp.pad(ew, (0, EPAD - E))
    row2 = rowp.reshape(ER, EB)
    col2 = colp.reshape(ER, EB)
    packed = jnp.stack([row2, col2], axis=1)

    dega, degb = _deg(col2, ewp)
    dis, dis2 = _rsqrt(dega.reshape(NPAD // 128, 128),
                       degb.reshape(NPAD // 128, 128))
    dis = dis.reshape(NPAD)
    dis2 = dis2.reshape(NPAD)
    normf = _norm(rowp, colp, ewp, dis)
    y = _mm1(x, W1)
    h = _agg1(*y, packed, normf, dis2, b1)
    hw = _mm2(*h, W2)
    o = _agg2(*hw, packed, normf, dis2, b2)
    return jnp.concatenate([o[0][:N], o[1][:N]], axis=1)


# speedup vs baseline: 5.9692x; 1.0394x over previous
import jax
import jax.numpy as jnp
from jax import lax
from jax.experimental import pallas as pl
from jax.experimental.pallas import tpu as pltpu
from jax.experimental.pallas import tpu_sc as plsc

N = 10000
NPAD = 10240
E = 160000
EPAD = 163840
L = 16
NC, NS = 2, 16
CH = 128
EB = 64
ER = EPAD // EB
RPT = NPAD // NS
NB = 64
ERPT = ER // NS
ERPW = ER // (NC * NS)

_mesh = plsc.VectorSubcoreMesh(core_axis_name="c", subcore_axis_name="s",
                               num_cores=NC, num_subcores=NS)
_sc_params = pltpu.CompilerParams(needs_layout_passes=False)


def _splat_i(v):
    return jnp.full((L,), v, jnp.int32)


def _deg_body(col2, ewf, dega_out, degb_out,
              deg_sh, colb, ewb, nodev):
    c = lax.axis_index("c")
    s = lax.axis_index("s")
    t0 = s * RPT
    e0 = (c * NS + s) * ERPW

    pltpu.sync_copy(col2.at[pl.ds(e0, ERPW)], colb)
    pltpu.sync_copy(ewf.at[pl.ds(e0 * EB, ERPW * EB)], ewb)

    init = jnp.where(c == 0, 1.0, 0.0).astype(jnp.float32)

    def _init(k, z):
        nodev[pl.ds(k * L, L)] = jnp.full((L,), 1.0, jnp.float32) * init
        return z
    lax.fori_loop(0, RPT // L, _init, 0)
    pltpu.sync_copy(nodev, deg_sh.at[pl.ds(t0, RPT)])
    plsc.subcore_barrier()

    def _scat(b, z):
        pltpu.sync_copy(ewb.at[pl.ds(b * EB, EB)],
                        deg_sh.at[colb.at[b]], add=True)
        return z
    lax.fori_loop(0, ERPW, _scat, 0)
    plsc.subcore_barrier()

    pltpu.sync_copy(deg_sh.at[pl.ds(t0, RPT)], nodev)

    @pl.when(c == 0)
    def _():
        pltpu.sync_copy(nodev, dega_out.at[pl.ds(t0, RPT)])

    @pl.when(c == 1)
    def _():
        pltpu.sync_copy(nodev, degb_out.at[pl.ds(t0, RPT)])


_deg = pl.kernel(
    _deg_body,
    out_type=(jax.ShapeDtypeStruct((NPAD,), jnp.float32),
              jax.ShapeDtypeStruct((NPAD,), jnp.float32)),
    mesh=_mesh,
    compiler_params=_sc_params,
    scratch_types=[
        pltpu.VMEM_SHARED((NPAD,), jnp.float32),
        pltpu.VMEM((ERPW, EB), jnp.int32),
        pltpu.VMEM((ERPW * EB,), jnp.float32),
        pltpu.VMEM((RPT,), jnp.float32),
    ],
)


def _rsqrt_body(da_ref, db_ref, dis_ref, dis2_ref):
    d = da_ref[...] + db_ref[...]
    r = lax.rsqrt(d)
    dis_ref[...] = r
    dis2_ref[...] = r * r


_rsqrt = pl.pallas_call(
    _rsqrt_body,
    out_shape=[jax.ShapeDtypeStruct((NPAD // 128, 128), jnp.float32)] * 2,
)


def _norm_body(rowf, colf, ewf, dis, normf,
               dis_full, rown, coln, ewn, normb):
    c = lax.axis_index("c")
    s = lax.axis_index("s")
    n0 = (c * NS + s) * ERPW * EB

    pltpu.sync_copy(dis, dis_full)
    pltpu.sync_copy(rowf.at[pl.ds(n0, ERPW * EB)], rown)
    pltpu.sync_copy(colf.at[pl.ds(n0, ERPW * EB)], coln)
    pltpu.sync_copy(ewf.at[pl.ds(n0, ERPW * EB)], ewn)

    def _nrm(i, z):
        sl = pl.ds(i * L, L)
        ri = rown[sl]
        ci = coln[sl]
        wv = ewn[sl]
        dr = plsc.load_gather(dis_full, [ri])
        dc = plsc.load_gather(dis_full, [ci])
        normb[sl] = dr * wv * dc
        return z
    lax.fori_loop(0, ERPW * EB // L, _nrm, 0)
    pltpu.sync_copy(normb, normf.at[pl.ds(n0, ERPW * EB)])


_norm = pl.kernel(
    _norm_body,
    out_type=jax.ShapeDtypeStruct((EPAD,), jnp.float32),
    mesh=_mesh,
    compiler_params=_sc_params,
    scratch_types=[
        pltpu.VMEM((NPAD,), jnp.float32),
        pltpu.VMEM((ERPW * EB,), jnp.int32),
        pltpu.VMEM((ERPW * EB,), jnp.int32),
        pltpu.VMEM((ERPW * EB,), jnp.float32),
        pltpu.VMEM((ERPW * EB,), jnp.float32),
    ],
)


def _make_agg(nch):
    rounds = nch // NC

    def _body(*refs):
        ys = refs[:nch]
        packed, normf, dis2, bias = refs[nch:nch + 4]
        outs = refs[nch + 4:nch + 4 + nch]
        rest = refs[nch + 4 + nch:]
        acc = rest[0]
        G = rest[1:5]
        S = rest[5:13]
        NRM = rest[13:21]
        dis2v, biasv = rest[21:23]
        sS = rest[23:31]
        gs = rest[31:35]
        sc = rest[35:39]
        G0 = G[0]

        c = lax.axis_index("c")
        s = lax.axis_index("s")
        t0 = s * RPT
        e0 = s * ERPT

        pltpu.sync_copy(dis2.at[pl.ds(t0, RPT)], dis2v)

        def _scale_rows(buf, nrows, base_fn):
            def _se(e, z):
                sp = base_fn(e)
                for k in range(CH // L):
                    sl = pl.ds(k * L, L)
                    buf[e, sl] = buf[e, sl] * sp
                return z
            lax.fori_loop(0, nrows, _se, 0)

        def _round(y, out, ch):
            def _didx(b, sl):
                return pltpu.make_async_copy(packed.at[e0 + b], S[sl], sS[sl])

            def _dnrm(b, sl):
                return pltpu.make_async_copy(
                    normf.at[pl.ds((e0 + b) * EB, EB)], NRM[sl], sS[sl])

            def _dgat(b, p, sl):
                return pltpu.make_async_copy(y.at[S[sl].at[0]], G[p], gs[p])

            def _dsct(p, sl):
                return pltpu.make_async_copy(G[p], acc.at[S[sl].at[1]], sc[p])

            pltpu.sync_copy(bias.at[pl.ds(ch * CH, CH)], biasv)
            for bb in range(RPT // NB):
                r0 = t0 + bb * NB
                pltpu.sync_copy(y.at[pl.ds(r0, NB)], G0)
                _scale_rows(G0, NB, lambda e, _bb=bb: plsc.load_gather(
                    dis2v, [_splat_i(_bb * NB + e)]))
                pltpu.sync_copy(G0, acc.at[pl.ds(r0, NB)])
            plsc.subcore_barrier()

            for k in range(6):
                _didx(k, k).start()
                _dnrm(k, k).start()
            for k in range(2):
                _didx(k, k).wait()
                _dnrm(k, k).wait()
                _dgat(k, k, k).start()

            def _step(d, z):
                for j in range(8):
                    b = d * 8 + j
                    if j < 2:
                        @pl.when(b >= 2)
                        def _():
                            _dsct((j + 2) & 3, (j + 6) & 7).wait()
                    else:
                        _dsct((j + 2) & 3, (j + 6) & 7).wait()

                    def _adv():
                        _didx(b + 2, (j + 2) & 7).wait()
                        _dnrm(b + 2, (j + 2) & 7).wait()
                        _dgat(b + 2, (j + 2) & 3, (j + 2) & 7).start()
                    if j >= 6:
                        pl.when(b + 2 < ERPT)(_adv)
                    else:
                        _adv()

                    def _pref():
                        _didx(b + 6, (j + 6) & 7).start()
                        _dnrm(b + 6, (j + 6) & 7).start()
                    if j >= 2:
                        pl.when(b + 6 < ERPT)(_pref)
                    else:
                        _pref()

                    _dgat(b, j & 3, j).wait()
                    _scale_rows(G[j & 3], EB,
                                lambda e, _j=j: plsc.load_gather(
                                    NRM[_j], [_splat_i(e)]))
                    _dsct(j & 3, j).start(add=True)
                return z
            lax.fori_loop(0, ERPT // 8, _step, 0)
            _dsct(2, 6).wait()
            _dsct(3, 7).wait()
            plsc.subcore_barrier()

            for bb in range(RPT // NB):
                r0 = t0 + bb * NB
                pltpu.sync_copy(acc.at[pl.ds(r0, NB)], G0)

                def _wo(e, z):
                    for k in range(CH // L):
                        sl = pl.ds(k * L, L)
                        G0[e, sl] = jnp.maximum(G0[e, sl] + biasv[sl], 0.0)
                    return z
                lax.fori_loop(0, NB, _wo, 0)
                pltpu.sync_copy(G0, out.at[pl.ds(r0, NB)])
            plsc.subcore_barrier()

        for core in range(NC):
            @pl.when(c == core)
            def _(core=core):
                for r in range(rounds):
                    ch = core * rounds + r
                    _round(ys[ch], outs[ch], ch)

    return pl.kernel(
        _body,
        out_type=tuple(jax.ShapeDtypeStruct((NPAD, CH), jnp.float32)
                       for _ in range(nch)),
        mesh=_mesh,
        compiler_params=_sc_params,
        scratch_types=(
            [pltpu.VMEM_SHARED((NPAD, CH), jnp.float32)]
            + [pltpu.VMEM((EB, CH), jnp.float32)] * 4
            + [pltpu.VMEM((2, EB), jnp.int32)] * 8
            + [pltpu.VMEM((EB,), jnp.float32)] * 8
            + [pltpu.VMEM((RPT,), jnp.float32)]
            + [pltpu.VMEM((CH,), jnp.float32)]
            + [pltpu.SemaphoreType.DMA] * 16
        ),
    )


_agg1 = _make_agg(4)
_agg2 = _make_agg(2)


def _mm1_body(x_ref, w_ref, *out_refs):
    acc = jnp.dot(x_ref[...], w_ref[...], preferred_element_type=jnp.float32)
    for ci in range(len(out_refs)):
        out_refs[ci][...] = acc[:, ci * CH:(ci + 1) * CH]


_mm1 = pl.pallas_call(
    _mm1_body,
    grid=(NPAD // 512,),
    in_specs=[pl.BlockSpec((512, 1728), lambda i: (i, 0)),
              pl.BlockSpec((1728, 512), lambda i: (0, 0))],
    out_specs=[pl.BlockSpec((512, CH), lambda i: (i, 0))] * 4,
    out_shape=[jax.ShapeDtypeStruct((NPAD, CH), jnp.float32)] * 4,
)


def _mm2_body(h0, h1, h2, h3, w_ref, *out_refs):
    h = jnp.concatenate([h0[...], h1[...], h2[...], h3[...]], axis=1)
    acc = jnp.dot(h, w_ref[...], preferred_element_type=jnp.float32)
    for ci in range(len(out_refs)):
        out_refs[ci][...] = acc[:, ci * CH:(ci + 1) * CH]


_mm2 = pl.pallas_call(
    _mm2_body,
    grid=(NPAD // 512,),
    in_specs=[pl.BlockSpec((512, CH), lambda i: (i, 0))] * 4 +
             [pl.BlockSpec((512, 256), lambda i: (0, 0))],
    out_specs=[pl.BlockSpec((512, CH), lambda i: (i, 0))] * 2,
    out_shape=[jax.ShapeDtypeStruct((NPAD, CH), jnp.float32)] * 2,
)


def kernel(x, edge_index, edge_weight, W1, b1, W2, b2):
    row = edge_index[0].astype(jnp.int32)
    col = edge_index[1].astype(jnp.int32)
    ew = edge_weight.astype(jnp.float32)
    rowp = jnp.pad(row, (0, EPAD - E))
    colp = jnp.pad(col, (0, EPAD - E))
    ewp = jnp.pad(ew, (0, EPAD - E))
    row2 = rowp.reshape(ER, EB)
    col2 = colp.reshape(ER, EB)
    packed = jnp.stack([row2, col2], axis=1)

    dega, degb = _deg(col2, ewp)
    dis, dis2 = _rsqrt(dega.reshape(NPAD // 128, 128),
                       degb.reshape(NPAD // 128, 128))
    dis = dis.reshape(NPAD)
    dis2 = dis2.reshape(NPAD)
    normf = _norm(rowp, colp, ewp, dis)
    y = _mm1(x, W1)
    h = _agg1(*y, packed, normf, dis2, b1)
    hw = _mm2(*h, W2)
    o = _agg2(*hw, packed, normf, dis2, b2)
    return jnp.concatenate([o[0][:N], o[1][:N]], axis=1)


# speedup vs baseline: 6.0728x; 1.0174x over previous
import jax
import jax.numpy as jnp
from jax import lax
from jax.experimental import pallas as pl
from jax.experimental.pallas import tpu as pltpu
from jax.experimental.pallas import tpu_sc as plsc

N = 10000
NPAD = 10240
E = 160000
EPAD = 163840
L = 16
NC, NS = 2, 16
CH = 128
EB = 80
ER = EPAD // EB
RPT = NPAD // NS
NB = 80
ERPT = ER // NS
ERPW = ER // (NC * NS)

_mesh = plsc.VectorSubcoreMesh(core_axis_name="c", subcore_axis_name="s",
                               num_cores=NC, num_subcores=NS)
_sc_params = pltpu.CompilerParams(needs_layout_passes=False)


def _splat_i(v):
    return jnp.full((L,), v, jnp.int32)


def _deg_body(col2, ewf, dega_out, degb_out,
              deg_sh, colb, ewb, nodev):
    c = lax.axis_index("c")
    s = lax.axis_index("s")
    t0 = s * RPT
    e0 = (c * NS + s) * ERPW

    pltpu.sync_copy(col2.at[pl.ds(e0, ERPW)], colb)
    pltpu.sync_copy(ewf.at[pl.ds(e0 * EB, ERPW * EB)], ewb)

    init = jnp.where(c == 0, 1.0, 0.0).astype(jnp.float32)

    def _init(k, z):
        nodev[pl.ds(k * L, L)] = jnp.full((L,), 1.0, jnp.float32) * init
        return z
    lax.fori_loop(0, RPT // L, _init, 0)
    pltpu.sync_copy(nodev, deg_sh.at[pl.ds(t0, RPT)])
    plsc.subcore_barrier()

    def _scat(b, z):
        pltpu.sync_copy(ewb.at[pl.ds(b * EB, EB)],
                        deg_sh.at[colb.at[b]], add=True)
        return z
    lax.fori_loop(0, ERPW, _scat, 0)
    plsc.subcore_barrier()

    pltpu.sync_copy(deg_sh.at[pl.ds(t0, RPT)], nodev)

    @pl.when(c == 0)
    def _():
        pltpu.sync_copy(nodev, dega_out.at[pl.ds(t0, RPT)])

    @pl.when(c == 1)
    def _():
        pltpu.sync_copy(nodev, degb_out.at[pl.ds(t0, RPT)])


_deg = pl.kernel(
    _deg_body,
    out_type=(jax.ShapeDtypeStruct((NPAD,), jnp.float32),
              jax.ShapeDtypeStruct((NPAD,), jnp.float32)),
    mesh=_mesh,
    compiler_params=_sc_params,
    scratch_types=[
        pltpu.VMEM_SHARED((NPAD,), jnp.float32),
        pltpu.VMEM((ERPW, EB), jnp.int32),
        pltpu.VMEM((ERPW * EB,), jnp.float32),
        pltpu.VMEM((RPT,), jnp.float32),
    ],
)


def _rsqrt_body(da_ref, db_ref, dis_ref, dis2_ref):
    d = da_ref[...] + db_ref[...]
    r = lax.rsqrt(d)
    dis_ref[...] = r
    dis2_ref[...] = r * r


_rsqrt = pl.pallas_call(
    _rsqrt_body,
    out_shape=[jax.ShapeDtypeStruct((NPAD // 128, 128), jnp.float32)] * 2,
)


def _norm_body(rowf, colf, ewf, dis, normf,
               dis_full, rown, coln, ewn, normb):
    c = lax.axis_index("c")
    s = lax.axis_index("s")
    n0 = (c * NS + s) * ERPW * EB

    pltpu.sync_copy(dis, dis_full)
    pltpu.sync_copy(rowf.at[pl.ds(n0, ERPW * EB)], rown)
    pltpu.sync_copy(colf.at[pl.ds(n0, ERPW * EB)], coln)
    pltpu.sync_copy(ewf.at[pl.ds(n0, ERPW * EB)], ewn)

    def _nrm(i, z):
        sl = pl.ds(i * L, L)
        ri = rown[sl]
        ci = coln[sl]
        wv = ewn[sl]
        dr = plsc.load_gather(dis_full, [ri])
        dc = plsc.load_gather(dis_full, [ci])
        normb[sl] = dr * wv * dc
        return z
    lax.fori_loop(0, ERPW * EB // L, _nrm, 0)
    pltpu.sync_copy(normb, normf.at[pl.ds(n0, ERPW * EB)])


_norm = pl.kernel(
    _norm_body,
    out_type=jax.ShapeDtypeStruct((EPAD,), jnp.float32),
    mesh=_mesh,
    compiler_params=_sc_params,
    scratch_types=[
        pltpu.VMEM((NPAD,), jnp.float32),
        pltpu.VMEM((ERPW * EB,), jnp.int32),
        pltpu.VMEM((ERPW * EB,), jnp.int32),
        pltpu.VMEM((ERPW * EB,), jnp.float32),
        pltpu.VMEM((ERPW * EB,), jnp.float32),
    ],
)


def _make_agg(nch):
    rounds = nch // NC

    def _body(*refs):
        ys = refs[:nch]
        packed, normf, dis2, bias = refs[nch:nch + 4]
        outs = refs[nch + 4:nch + 4 + nch]
        rest = refs[nch + 4 + nch:]
        acc = rest[0]
        G = rest[1:5]
        S = rest[5:13]
        NRM = rest[13:21]
        dis2v, biasv = rest[21:23]
        sS = rest[23:31]
        gs = rest[31:35]
        sc = rest[35:39]
        G0 = G[0]

        c = lax.axis_index("c")
        s = lax.axis_index("s")
        t0 = s * RPT
        e0 = s * ERPT

        pltpu.sync_copy(dis2.at[pl.ds(t0, RPT)], dis2v)

        def _scale_rows(buf, nrows, base_fn):
            def _se(e, z):
                sp = base_fn(e)
                for k in range(CH // L):
                    sl = pl.ds(k * L, L)
                    buf[e, sl] = buf[e, sl] * sp
                return z
            lax.fori_loop(0, nrows, _se, 0)

        def _round(y, out, ch):
            def _didx(b, sl):
                return pltpu.make_async_copy(packed.at[e0 + b], S[sl], sS[sl])

            def _dnrm(b, sl):
                return pltpu.make_async_copy(
                    normf.at[pl.ds((e0 + b) * EB, EB)], NRM[sl], sS[sl])

            def _dgat(b, p, sl):
                return pltpu.make_async_copy(y.at[S[sl].at[0]], G[p], gs[p])

            def _dsct(p, sl):
                return pltpu.make_async_copy(G[p], acc.at[S[sl].at[1]], sc[p])

            pltpu.sync_copy(bias.at[pl.ds(ch * CH, CH)], biasv)
            for bb in range(RPT // NB):
                r0 = t0 + bb * NB
                pltpu.sync_copy(y.at[pl.ds(r0, NB)], G0)
                _scale_rows(G0, NB, lambda e, _bb=bb: plsc.load_gather(
                    dis2v, [_splat_i(_bb * NB + e)]))
                pltpu.sync_copy(G0, acc.at[pl.ds(r0, NB)])
            plsc.subcore_barrier()

            for k in range(6):
                _didx(k, k).start()
                _dnrm(k, k).start()
            for k in range(2):
                _didx(k, k).wait()
                _dnrm(k, k).wait()
                _dgat(k, k, k).start()

            def _step(d, z):
                for j in range(8):
                    b = d * 8 + j
                    if j < 2:
                        @pl.when(b >= 2)
                        def _():
                            _dsct((j + 2) & 3, (j + 6) & 7).wait()
                    else:
                        _dsct((j + 2) & 3, (j + 6) & 7).wait()

                    def _adv():
                        _didx(b + 2, (j + 2) & 7).wait()
                        _dnrm(b + 2, (j + 2) & 7).wait()
                        _dgat(b + 2, (j + 2) & 3, (j + 2) & 7).start()
                    if j >= 6:
                        pl.when(b + 2 < ERPT)(_adv)
                    else:
                        _adv()

                    def _pref():
                        _didx(b + 6, (j + 6) & 7).start()
                        _dnrm(b + 6, (j + 6) & 7).start()
                    if j >= 2:
                        pl.when(b + 6 < ERPT)(_pref)
                    else:
                        _pref()

                    _dgat(b, j & 3, j).wait()
                    _scale_rows(G[j & 3], EB,
                                lambda e, _j=j: plsc.load_gather(
                                    NRM[_j], [_splat_i(e)]))
                    _dsct(j & 3, j).start(add=True)
                return z
            lax.fori_loop(0, ERPT // 8, _step, 0)
            _dsct(2, 6).wait()
            _dsct(3, 7).wait()
            plsc.subcore_barrier()

            for bb in range(RPT // NB):
                r0 = t0 + bb * NB
                pltpu.sync_copy(acc.at[pl.ds(r0, NB)], G0)

                def _wo(e, z):
                    for k in range(CH // L):
                        sl = pl.ds(k * L, L)
                        G0[e, sl] = jnp.maximum(G0[e, sl] + biasv[sl], 0.0)
                    return z
                lax.fori_loop(0, NB, _wo, 0)
                pltpu.sync_copy(G0, out.at[pl.ds(r0, NB)])
            plsc.subcore_barrier()

        for core in range(NC):
            @pl.when(c == core)
            def _(core=core):
                for r in range(rounds):
                    ch = core * rounds + r
                    _round(ys[ch], outs[ch], ch)

    return pl.kernel(
        _body,
        out_type=tuple(jax.ShapeDtypeStruct((NPAD, CH), jnp.float32)
                       for _ in range(nch)),
        mesh=_mesh,
        compiler_params=_sc_params,
        scratch_types=(
            [pltpu.VMEM_SHARED((NPAD, CH), jnp.float32)]
            + [pltpu.VMEM((EB, CH), jnp.float32)] * 4
            + [pltpu.VMEM((2, EB), jnp.int32)] * 8
            + [pltpu.VMEM((EB,), jnp.float32)] * 8
            + [pltpu.VMEM((RPT,), jnp.float32)]
            + [pltpu.VMEM((CH,), jnp.float32)]
            + [pltpu.SemaphoreType.DMA] * 16
        ),
    )


_agg1 = _make_agg(4)
_agg2 = _make_agg(2)


def _mm1_body(x_ref, w_ref, *out_refs):
    acc = jnp.dot(x_ref[...], w_ref[...], preferred_element_type=jnp.float32)
    for ci in range(len(out_refs)):
        out_refs[ci][...] = acc[:, ci * CH:(ci + 1) * CH]


_mm1 = pl.pallas_call(
    _mm1_body,
    grid=(NPAD // 512,),
    in_specs=[pl.BlockSpec((512, 1728), lambda i: (i, 0)),
              pl.BlockSpec((1728, 512), lambda i: (0, 0))],
    out_specs=[pl.BlockSpec((512, CH), lambda i: (i, 0))] * 4,
    out_shape=[jax.ShapeDtypeStruct((NPAD, CH), jnp.float32)] * 4,
)


def _mm2_body(h0, h1, h2, h3, w_ref, *out_refs):
    h = jnp.concatenate([h0[...], h1[...], h2[...], h3[...]], axis=1)
    acc = jnp.dot(h, w_ref[...], preferred_element_type=jnp.float32)
    for ci in range(len(out_refs)):
        out_refs[ci][...] = acc[:, ci * CH:(ci + 1) * CH]


_mm2 = pl.pallas_call(
    _mm2_body,
    grid=(NPAD // 512,),
    in_specs=[pl.BlockSpec((512, CH), lambda i: (i, 0))] * 4 +
             [pl.BlockSpec((512, 256), lambda i: (0, 0))],
    out_specs=[pl.BlockSpec((512, CH), lambda i: (i, 0))] * 2,
    out_shape=[jax.ShapeDtypeStruct((NPAD, CH), jnp.float32)] * 2,
)


def kernel(x, edge_index, edge_weight, W1, b1, W2, b2):
    row = edge_index[0].astype(jnp.int32)
    col = edge_index[1].astype(jnp.int32)
    ew = edge_weight.astype(jnp.float32)
    rowp = jnp.pad(row, (0, EPAD - E))
    colp = jnp.pad(col, (0, EPAD - E))
    ewp = jnp.pad(ew, (0, EPAD - E))
    row2 = rowp.reshape(ER, EB)
    col2 = colp.reshape(ER, EB)
    packed = jnp.stack([row2, col2], axis=1)

    dega, degb = _deg(col2, ewp)
    dis, dis2 = _rsqrt(dega.reshape(NPAD // 128, 128),
                       degb.reshape(NPAD // 128, 128))
    dis = dis.reshape(NPAD)
    dis2 = dis2.reshape(NPAD)
    normf = _norm(rowp, colp, ewp, dis)
    y = _mm1(x, W1)
    h = _agg1(*y, packed, normf, dis2, b1)
    hw = _mm2(*h, W2)
    o = _agg2(*hw, packed, normf, dis2, b2)
    return jnp.concatenate([o[0][:N], o[1][:N]], axis=1)
